# baseline (speedup 1.0000x reference)
import functools

import numpy as np
import jax
import jax.numpy as jnp
from jax import lax
from jax.experimental import pallas as pl
from jax.experimental.pallas import tpu as pltpu
from jax.experimental.pallas import tpu_sc as plsc

_PATCH = (1, 4, 16, 64, 256, 1024)
_B, _C, _L, _V = 16, 64, 1024, 8192
_NPHI = 4
_RESI = 0.5
_BETA = 0.25
_VT = 128
_RC = 1024
_CP = 128

_TICKS = np.linspace(1.0 / 3.0 / _NPHI, 1.0 - 1.0 / 3.0 / _NPHI, _NPHI)
_PHI_IDX = tuple(
    int(np.argmin(np.abs(_TICKS - si / (len(_PATCH) - 1))))
    for si in range(len(_PATCH))
)
_HI = lax.Precision.HIGHEST


def _dot3(m, x):
    x1 = x.astype(jnp.bfloat16).astype(jnp.float32)
    d1 = x - x1
    x2 = d1.astype(jnp.bfloat16).astype(jnp.float32)
    x3 = d1 - x2
    acc = jnp.dot(m, x1, preferred_element_type=jnp.float32)
    acc = acc + jnp.dot(m, x2, preferred_element_type=jnp.float32)
    return acc + jnp.dot(m, x3, preferred_element_type=jnp.float32)


def _interp_matrix(l_in, l_out):
    scale = l_in / l_out
    coords = (np.arange(l_out, dtype=np.float32) + np.float32(0.5)) * np.float32(scale) - np.float32(0.5)
    coords = np.clip(coords, 0.0, l_in - 1).astype(np.float32)
    lo = np.floor(coords).astype(np.int32)
    hi = np.minimum(lo + 1, l_in - 1)
    w = (coords - lo.astype(np.float32)).astype(np.float32)
    m = np.zeros((l_out, l_in), dtype=np.float32)
    m[np.arange(l_out), lo] += (1.0 - w).astype(np.float32)
    m[np.arange(l_out), hi] += w
    return m


def _prep_body(w_ref, wn_ref, wp_ref, w1_ref, w2_ref, w3_ref):
    w = w_ref[...]
    ss = jnp.sum(w * w, axis=1, keepdims=True)
    wn_ref[...] = w / jnp.maximum(jnp.sqrt(ss), 1e-12)
    wp_ref[:, :_C] = w
    wp_ref[:, _C:] = jnp.zeros_like(w)
    w1 = w.astype(jnp.bfloat16)
    d1 = w - w1.astype(jnp.float32)
    w2 = d1.astype(jnp.bfloat16)
    w1_ref[...] = w1
    w2_ref[...] = w2
    w3_ref[...] = d1 - w2.astype(jnp.float32)


def _prep_codebook(W):
    return pl.pallas_call(
        _prep_body,
        grid=(_V // 1024,),
        in_specs=[pl.BlockSpec((1024, _C), lambda v: (v, 0))],
        out_specs=[pl.BlockSpec((1024, _C), lambda v: (v, 0)),
                   pl.BlockSpec((1024, _CP), lambda v: (v, 0)),
                   pl.BlockSpec((1024, _C), lambda v: (v, 0)),
                   pl.BlockSpec((1024, _C), lambda v: (v, 0)),
                   pl.BlockSpec((1024, _C), lambda v: (v, 0))],
        out_shape=[jax.ShapeDtypeStruct((_V, _C), jnp.float32),
                   jax.ShapeDtypeStruct((_V, _CP), jnp.float32),
                   jax.ShapeDtypeStruct((_V, _C), jnp.bfloat16),
                   jax.ShapeDtypeStruct((_V, _C), jnp.bfloat16),
                   jax.ShapeDtypeStruct((_V, _C), jnp.float32)],
    )(W)


def _norm_rows(rp):
    nrm = jnp.sqrt(jnp.sum(rp * rp, axis=1, keepdims=True))
    return rp / jnp.maximum(nrm, 1e-12)


def _vscan(rows, wn_ref):
    n = rows.shape[0]
    io0 = lax.broadcasted_iota(jnp.int32, (_VT, n), 0)
    big = jnp.int32(2 ** 30)
    bv = bi = None
    for i in range(_V // _VT):
        wt = wn_ref[i * _VT:(i + 1) * _VT, :]
        s = lax.dot_general(wt, rows, (((1,), (1,)), ((), ())),
                            preferred_element_type=jnp.float32)
        m = jnp.max(s, axis=0, keepdims=True)
        ci = jnp.min(jnp.where(s == m, io0, big), axis=0, keepdims=True) + i * _VT
        if bv is None:
            bv, bi = m, ci
        else:
            upd = m > bv
            bv = jnp.where(upd, m, bv)
            bi = jnp.where(upd, ci, bi)
    return bi


def _argmax_small_body(pl_, first, *refs):
    if first:
        g_ref, d_ref, wn_ref, idx_ref = refs
    else:
        g_ref, fh_ref, d_ref, wn_ref, idx_ref = refs
    d = d_ref[...]
    parts = []
    for b in range(_B):
        rest = g_ref[b] if first else g_ref[b] - fh_ref[b]
        parts.append(_dot3(d, rest))
    rp = jnp.concatenate(parts, axis=0)
    bi = _vscan(_norm_rows(rp), wn_ref)
    n = bi.shape[1]
    npad = max(256, n)
    if n < npad:
        bi = jnp.concatenate(
            [bi, jnp.zeros((1, npad - n), jnp.int32)], axis=1)
    idx_ref[0] = bi


def _argmax_mid_body(g_ref, fh_ref, d_ref, wn_ref, idx_ref):
    rest = g_ref[0] - fh_ref[0]
    rp = _dot3(d_ref[...], rest)
    idx_ref[0] = _vscan(_norm_rows(rp), wn_ref)


def _argmax_last_body(g_ref, fh_ref, wn_ref, idx_ref):
    rest = g_ref[...] - fh_ref[...]
    idx_ref[0] = _vscan(_norm_rows(rest), wn_ref)


def _nearest_codes(si, g, f_hat, Wn, d_mat):
    pl_ = _PATCH[si]
    n = _B * pl_
    if si == len(_PATCH) - 1:
        idx = pl.pallas_call(
            _argmax_last_body,
            grid=(n // _RC,),
            in_specs=[pl.BlockSpec((_RC, _C), lambda c: (c, 0)),
                      pl.BlockSpec((_RC, _C), lambda c: (c, 0)),
                      pl.BlockSpec((_V, _C), lambda c: (0, 0))],
            out_specs=pl.BlockSpec((1, 1, _RC), lambda c: (c, 0, 0)),
            out_shape=jax.ShapeDtypeStruct((n // _RC, 1, _RC), jnp.int32),
        )(g.reshape(n, _C), f_hat.reshape(n, _C), Wn)
    elif pl_ >= _RC:
        idx = pl.pallas_call(
            _argmax_mid_body,
            grid=(_B,),
            in_specs=[pl.BlockSpec((1, _L, _C), lambda b: (b, 0, 0)),
                      pl.BlockSpec((1, _L, _C), lambda b: (b, 0, 0)),
                      pl.BlockSpec((pl_, _L), lambda b: (0, 0)),
                      pl.BlockSpec((_V, _C), lambda b: (0, 0))],
            out_specs=pl.BlockSpec((1, 1, pl_), lambda b: (b, 0, 0)),
            out_shape=jax.ShapeDtypeStruct((_B, 1, pl_), jnp.int32),
        )(g, f_hat, d_mat, Wn)
    else:
        first = si == 0
        in_specs = [pl.BlockSpec((_B, _L, _C), lambda: (0, 0, 0))]
        args = [g]
        if not first:
            in_specs.append(pl.BlockSpec((_B, _L, _C), lambda: (0, 0, 0)))
            args.append(f_hat)
        in_specs += [pl.BlockSpec((pl_, _L), lambda: (0, 0)),
                     pl.BlockSpec((_V, _C), lambda: (0, 0))]
        args += [d_mat, Wn]
        idx = pl.pallas_call(
            functools.partial(_argmax_small_body, pl_, first),
            in_specs=in_specs,
            out_specs=pl.BlockSpec((1, 1, max(256, n)), lambda: (0, 0, 0)),
            out_shape=jax.ShapeDtypeStruct((1, 1, max(256, n)), jnp.int32),
        )(*args)
        return idx.reshape(max(256, n))
    return idx.reshape(n)


def _interp_lohi(l_in, l_out):
    scale = l_in / l_out
    coords = (np.arange(l_out, dtype=np.float32) + np.float32(0.5)) * np.float32(scale) - np.float32(0.5)
    coords = np.clip(coords, 0.0, l_in - 1).astype(np.float32)
    lo = np.floor(coords).astype(np.int32)
    hi = np.minimum(lo + 1, l_in - 1)
    w = (coords - lo.astype(np.float32)).astype(np.float32)
    return lo, hi, w


def _runs(vals):
    out = []
    for v in vals:
        if out and out[-1][0] == v:
            out[-1][1] += 1
        else:
            out.append([int(v), 1])
    return out


def _bcast_rows(hb, runs):
    return jnp.concatenate(
        [jnp.broadcast_to(hb[v:v + 1], (cnt, _C)) for v, cnt in runs], axis=0)


def _onehot_gather(bi, w1_ref, w2_ref, w3_ref):
    n = bi.shape[1]
    io0 = lax.broadcasted_iota(jnp.int32, (_VT, n), 0)
    h = None
    for i in range(_V // _VT):
        oh = jnp.where(io0 == (bi - i * _VT), 1.0, 0.0).astype(jnp.bfloat16)
        sl = slice(i * _VT, (i + 1) * _VT)
        part = lax.dot_general(oh, w1_ref[sl, :], (((0,), (0,)), ((), ())),
                               preferred_element_type=jnp.float32)
        part = part + lax.dot_general(oh, w2_ref[sl, :], (((0,), (0,)), ((), ())),
                                      preferred_element_type=jnp.float32)
        part = part + lax.dot_general(
            oh.astype(jnp.float32), w3_ref[sl, :], (((0,), (0,)), ((), ())),
            preferred_element_type=jnp.float32)
        h = part if h is None else h + part
    return h


def _mono_body(si, *refs):
    pl_ = _PATCH[si]
    first = si == 0
    i = 0
    g_ref = refs[i]; i += 1
    if not first:
        fh_ref = refs[i]; i += 1
        loss_ref = refs[i]; i += 1
    d_ref = refs[i]; i += 1
    wn_ref = refs[i]; i += 1
    w1_ref = refs[i]; i += 1
    w2_ref = refs[i]; i += 1
    w3_ref = refs[i]; i += 1
    if si == 1:
        wa_ref = refs[i]; i += 1
        wb_ref = refs[i]; i += 1
    elif si == 2:
        u_ref = refs[i]; i += 1
    k_ref = refs[i]; i += 1
    b_ref = refs[i]; i += 1
    fh_out, loss_out = refs[i], refs[i + 1]

    d = d_ref[...]
    parts = []
    for b in range(_B):
        rest = g_ref[b] if first else g_ref[b] - fh_ref[b]
        parts.append(_dot3(d, rest))
    rp = jnp.concatenate(parts, axis=0)
    bi = _vscan(_norm_rows(rp), wn_ref)
    h = _onehot_gather(bi, w1_ref, w2_ref, w3_ref)

    if si == 1:
        lo, hi, _ = _interp_lohi(pl_, _L)
        lo_runs, hi_runs = _runs(lo), _runs(hi)
    z = jnp.zeros((1, _C), jnp.float32)
    for b in range(_B):
        hb = h[b * pl_:(b + 1) * pl_]
        if si == 0:
            hu = jnp.broadcast_to(hb[0:1], (_L, _C))
        elif si == 1:
            hu = (wa_ref[...] * _bcast_rows(hb, lo_runs)
                  + wb_ref[...] * _bcast_rows(hb, hi_runs))
        else:
            hu = _dot3(u_ref[...], hb)
        prev = jnp.concatenate([z, hu[:-1]], axis=0)
        nxt = jnp.concatenate([hu[1:], z], axis=0)
        y = (jnp.dot(prev, k_ref[0], preferred_element_type=jnp.float32)
             + jnp.dot(hu, k_ref[1], preferred_element_type=jnp.float32)
             + jnp.dot(nxt, k_ref[2], preferred_element_type=jnp.float32)
             + b_ref[...])
        ho = (1.0 - _RESI) * hu + _RESI * y
        fh = ho if first else fh_ref[b] + ho
        fh_out[b] = fh
        dd = fh - g_ref[b]
        part = jnp.sum(dd * dd).reshape(1, 1)
        loss_out[b] = part if first else loss_ref[b] + part


def _mono_scale(si, g, f_hat, loss, d_mat, Wn, W1, W2, W3, u_mat, k_mats, bias):
    pl_ = _PATCH[si]
    first = si == 0
    full3 = pl.BlockSpec((_B, _L, _C), lambda: (0, 0, 0))
    in_specs = [full3]
    args = [g]
    aliases = {}
    if not first:
        in_specs += [full3, pl.BlockSpec((_B, 1, 1), lambda: (0, 0, 0))]
        args += [f_hat, loss]
    in_specs += [pl.BlockSpec((pl_, _L), lambda: (0, 0)),
                 pl.BlockSpec((_V, _C), lambda: (0, 0)),
                 pl.BlockSpec((_V, _C), lambda: (0, 0)),
                 pl.BlockSpec((_V, _C), lambda: (0, 0)),
                 pl.BlockSpec((_V, _C), lambda: (0, 0))]
    args += [d_mat, Wn, W1, W2, W3]
    if si == 1:
        lo, hi, w = _interp_lohi(pl_, _L)
        wa = jnp.asarray((1.0 - w).reshape(_L, 1))
        wb = jnp.asarray(w.reshape(_L, 1))
        in_specs += [pl.BlockSpec((_L, 1), lambda: (0, 0)),
                     pl.BlockSpec((_L, 1), lambda: (0, 0))]
        args += [wa, wb]
    elif si == 2:
        in_specs.append(pl.BlockSpec((_L, pl_), lambda: (0, 0)))
        args.append(u_mat)
    in_specs += [pl.BlockSpec((3, _C, _C), lambda: (0, 0, 0)),
                 pl.BlockSpec((1, _C), lambda: (0, 0))]
    args += [k_mats, bias]
    if not first:
        aliases = {1: 0, 2: 1}
    return pl.pallas_call(
        functools.partial(_mono_body, si),
        in_specs=in_specs,
        out_specs=[full3, pl.BlockSpec((_B, 1, 1), lambda: (0, 0, 0))],
        out_shape=[jax.ShapeDtypeStruct((_B, _L, _C), jnp.float32),
                   jax.ShapeDtypeStruct((_B, 1, 1), jnp.float32)],
        input_output_aliases=aliases,
    )(*args)


@functools.cache
def _sc_gather_fn(npad):
    info = plsc.get_sparse_core_info()
    nw = info.num_cores * info.num_subcores
    bpw = npad // nw
    mesh = plsc.VectorSubcoreMesh(core_axis_name="c", subcore_axis_name="s")

    @functools.partial(
        pl.kernel,
        mesh=mesh,
        out_type=jax.ShapeDtypeStruct((npad, _CP), jnp.float32),
        scratch_types=[
            pltpu.VMEM((bpw,), jnp.int32),
            pltpu.VMEM((bpw, _CP), jnp.float32),
            pltpu.SemaphoreType.DMA,
        ],
    )
    def gather(table_hbm, idx_hbm, out_hbm, idx_v, rows_v, sem):
        wid = lax.axis_index("s") * info.num_cores + lax.axis_index("c")
        base = wid * bpw
        pltpu.sync_copy(idx_hbm.at[pl.ds(base, bpw)], idx_v)
        pltpu.async_copy(table_hbm.at[idx_v], rows_v, sem).wait()
        pltpu.sync_copy(rows_v, out_hbm.at[pl.ds(base, bpw)])

    return gather


def _gather_rows(Wp, idx):
    n = idx.shape[0]
    npad = max(256, n)
    if npad != n:
        idx = jnp.pad(idx, (0, npad - n))
    rows = _sc_gather_fn(npad)(Wp, idx)
    return rows[:n]


_CB = 4


def _compose_body(si, first, *refs):
    last = si == len(_PATCH) - 1
    i = 0
    h_ref = refs[i]; i += 1
    if not last:
        u_ref = refs[i]; i += 1
    k_ref = refs[i]; i += 1
    b_ref = refs[i]; i += 1
    g_ref = refs[i]; i += 1
    if not first:
        fh_ref = refs[i]; i += 1
        loss_ref = refs[i]; i += 1
    fh_out, loss_out = refs[i], refs[i + 1]

    z = jnp.zeros((1, _C), jnp.float32)
    for bb in range(_CB):
        hp = h_ref[bb][:, :_C]
        if last:
            hu = hp
        elif si == 1:
            hu = jnp.dot(u_ref[...], hp, preferred_element_type=jnp.float32,
                         precision=_HI)
        else:
            hu = _dot3(u_ref[...], hp)
        prev = jnp.concatenate([z, hu[:-1]], axis=0)
        nxt = jnp.concatenate([hu[1:], z], axis=0)
        y = (jnp.dot(prev, k_ref[0], preferred_element_type=jnp.float32)
             + jnp.dot(hu, k_ref[1], preferred_element_type=jnp.float32)
             + jnp.dot(nxt, k_ref[2], preferred_element_type=jnp.float32)
             + b_ref[...])
        ho = (1.0 - _RESI) * hu + _RESI * y
        fh = ho if first else fh_ref[bb] + ho
        fh_out[bb] = fh
        d = fh - g_ref[bb]
        part = jnp.sum(d * d).reshape(1, 1)
        loss_out[bb] = part if first else loss_ref[bb] + part


def _compose_scale(si, h_rows, u_mat, k_mats, bias, g, f_hat, loss):
    pl_ = _PATCH[si]
    last = si == len(_PATCH) - 1
    first = si == 0
    h = h_rows.reshape(_B, pl_, _CP)
    in_specs = [pl.BlockSpec((_CB, pl_, _CP), lambda b: (b, 0, 0))]
    args = [h]
    if not last:
        in_specs.append(pl.BlockSpec((_L, pl_), lambda b: (0, 0)))
        args.append(u_mat)
    in_specs += [
        pl.BlockSpec((3, _C, _C), lambda b: (0, 0, 0)),
        pl.BlockSpec((1, _C), lambda b: (0, 0)),
        pl.BlockSpec((_CB, _L, _C), lambda b: (b, 0, 0)),
    ]
    args += [k_mats, bias, g]
    aliases = {}
    if not first:
        in_specs += [pl.BlockSpec((_CB, _L, _C), lambda b: (b, 0, 0)),
                     pl.BlockSpec((_CB, 1, 1), lambda b: (b, 0, 0))]
        args += [f_hat, loss]
        aliases = {len(args) - 2: 0, len(args) - 1: 1}
    out = pl.pallas_call(
        functools.partial(_compose_body, si, first),
        grid=(_B // _CB,),
        in_specs=in_specs,
        out_specs=[
            pl.BlockSpec((_CB, _L, _C), lambda b: (b, 0, 0)),
            pl.BlockSpec((_CB, 1, 1), lambda b: (b, 0, 0)),
        ],
        out_shape=[
            jax.ShapeDtypeStruct((_B, _L, _C), jnp.float32),
            jax.ShapeDtypeStruct((_B, 1, 1), jnp.float32),
        ],
        input_output_aliases=aliases,
    )(*args)
    return out


def _loss_total_body(loss_ref, out_ref):
    s = jnp.sum(loss_ref[...])
    scale = (1.0 + _BETA) / (len(_PATCH) * _B * _C * _L)
    out_ref[...] = (s * scale).reshape(1, 1)


def _loss_total(loss):
    out = pl.pallas_call(
        _loss_total_body,
        in_specs=[pl.BlockSpec((_B, 1, 1), lambda: (0, 0, 0))],
        out_specs=pl.BlockSpec((1, 1), lambda: (0, 0)),
        out_shape=jax.ShapeDtypeStruct((1, 1), jnp.float32),
    )(loss)
    return out[0, 0]


def kernel(f_BCL, W, phi_w, phi_b):
    sn = len(_PATCH)
    g = jnp.transpose(f_BCL, (0, 2, 1))
    Wn, Wp, W1, W2, W3 = _prep_codebook(W)

    d_mats = [jnp.asarray(_interp_matrix(_L, p)) for p in _PATCH[:-1]]
    u_mats = [jnp.asarray(_interp_matrix(p, _L)) for p in _PATCH[:-1]]
    k_all = jnp.transpose(phi_w, (0, 3, 2, 1))

    f_hat = None
    loss = None
    for si in range(3):
        k = _PHI_IDX[si]
        f_hat, loss = _mono_scale(
            si, g, f_hat, loss, d_mats[si], Wn, W1, W2, W3,
            u_mats[si], k_all[k], phi_b[k].reshape(1, _C))
    for si in range(3, sn):
        idx = _nearest_codes(si, g, f_hat, Wn,
                             d_mats[si] if si < sn - 1 else None)
        h_rows = _gather_rows(Wp, idx)[:_B * _PATCH[si]]
        k = _PHI_IDX[si]
        f_hat, loss = _compose_scale(
            si, h_rows, u_mats[si] if si < sn - 1 else None,
            k_all[k], phi_b[k].reshape(1, _C), g, f_hat, loss)

    f_hat_out = jnp.transpose(f_hat, (0, 2, 1))
    return f_hat_out, _loss_total(loss)


# speedup vs baseline: 1.0684x; 1.0684x over previous
import functools

import numpy as np
import jax
import jax.numpy as jnp
from jax import lax
from jax.experimental import pallas as pl
from jax.experimental.pallas import tpu as pltpu
from jax.experimental.pallas import tpu_sc as plsc

_PATCH = (1, 4, 16, 64, 256, 1024)
_B, _C, _L, _V = 16, 64, 1024, 8192
_NPHI = 4
_RESI = 0.5
_BETA = 0.25
_VT = 256
_RC = 1024
_CP = 128

_TICKS = np.linspace(1.0 / 3.0 / _NPHI, 1.0 - 1.0 / 3.0 / _NPHI, _NPHI)
_PHI_IDX = tuple(
    int(np.argmin(np.abs(_TICKS - si / (len(_PATCH) - 1))))
    for si in range(len(_PATCH))
)
_HI = lax.Precision.HIGHEST


def _dot3(m, x):
    x1 = x.astype(jnp.bfloat16).astype(jnp.float32)
    d1 = x - x1
    x2 = d1.astype(jnp.bfloat16).astype(jnp.float32)
    x3 = d1 - x2
    acc = jnp.dot(m, x1, preferred_element_type=jnp.float32)
    acc = acc + jnp.dot(m, x2, preferred_element_type=jnp.float32)
    return acc + jnp.dot(m, x3, preferred_element_type=jnp.float32)


def _interp_matrix(l_in, l_out):
    scale = l_in / l_out
    coords = (np.arange(l_out, dtype=np.float32) + np.float32(0.5)) * np.float32(scale) - np.float32(0.5)
    coords = np.clip(coords, 0.0, l_in - 1).astype(np.float32)
    lo = np.floor(coords).astype(np.int32)
    hi = np.minimum(lo + 1, l_in - 1)
    w = (coords - lo.astype(np.float32)).astype(np.float32)
    m = np.zeros((l_out, l_in), dtype=np.float32)
    m[np.arange(l_out), lo] += (1.0 - w).astype(np.float32)
    m[np.arange(l_out), hi] += w
    return m


def _prep_body(w_ref, wn_ref, wp_ref, w1_ref, w2_ref, w3_ref):
    w = w_ref[...]
    ss = jnp.sum(w * w, axis=1, keepdims=True)
    wn_ref[...] = w / jnp.maximum(jnp.sqrt(ss), 1e-12)
    wp_ref[:, :_C] = w
    wp_ref[:, _C:] = jnp.zeros_like(w)
    w1 = w.astype(jnp.bfloat16)
    d1 = w - w1.astype(jnp.float32)
    w2 = d1.astype(jnp.bfloat16)
    w1_ref[...] = w1
    w2_ref[...] = w2
    w3_ref[...] = d1 - w2.astype(jnp.float32)


def _prep_codebook(W):
    return pl.pallas_call(
        _prep_body,
        grid=(_V // 1024,),
        in_specs=[pl.BlockSpec((1024, _C), lambda v: (v, 0))],
        out_specs=[pl.BlockSpec((1024, _C), lambda v: (v, 0)),
                   pl.BlockSpec((1024, _CP), lambda v: (v, 0)),
                   pl.BlockSpec((1024, _C), lambda v: (v, 0)),
                   pl.BlockSpec((1024, _C), lambda v: (v, 0)),
                   pl.BlockSpec((1024, _C), lambda v: (v, 0))],
        out_shape=[jax.ShapeDtypeStruct((_V, _C), jnp.float32),
                   jax.ShapeDtypeStruct((_V, _CP), jnp.float32),
                   jax.ShapeDtypeStruct((_V, _C), jnp.bfloat16),
                   jax.ShapeDtypeStruct((_V, _C), jnp.bfloat16),
                   jax.ShapeDtypeStruct((_V, _C), jnp.float32)],
    )(W)


def _norm_rows(rp):
    nrm = jnp.sqrt(jnp.sum(rp * rp, axis=1, keepdims=True))
    return rp / jnp.maximum(nrm, 1e-12)


def _vscan(rows, wn_ref):
    n = rows.shape[0]
    io0 = lax.broadcasted_iota(jnp.int32, (_VT, n), 0)
    big = jnp.int32(2 ** 30)
    bv = bi = None
    for i in range(_V // _VT):
        wt = wn_ref[i * _VT:(i + 1) * _VT, :]
        s = lax.dot_general(wt, rows, (((1,), (1,)), ((), ())),
                            preferred_element_type=jnp.float32)
        m = jnp.max(s, axis=0, keepdims=True)
        ci = jnp.min(jnp.where(s == m, io0, big), axis=0, keepdims=True) + i * _VT
        if bv is None:
            bv, bi = m, ci
        else:
            upd = m > bv
            bv = jnp.where(upd, m, bv)
            bi = jnp.where(upd, ci, bi)
    return bi


def _argmax_small_body(pl_, first, *refs):
    if first:
        g_ref, d_ref, wn_ref, idx_ref = refs
    else:
        g_ref, fh_ref, d_ref, wn_ref, idx_ref = refs
    d = d_ref[...]
    parts = []
    for b in range(_B):
        rest = g_ref[b] if first else g_ref[b] - fh_ref[b]
        parts.append(_dot3(d, rest))
    rp = jnp.concatenate(parts, axis=0)
    bi = _vscan(_norm_rows(rp), wn_ref)
    n = bi.shape[1]
    npad = max(256, n)
    if n < npad:
        bi = jnp.concatenate(
            [bi, jnp.zeros((1, npad - n), jnp.int32)], axis=1)
    idx_ref[0] = bi


def _argmax_mid_body(g_ref, fh_ref, d_ref, wn_ref, idx_ref):
    rest = g_ref[0] - fh_ref[0]
    rp = _dot3(d_ref[...], rest)
    idx_ref[0] = _vscan(_norm_rows(rp), wn_ref)


def _argmax_last_body(g_ref, fh_ref, wn_ref, idx_ref):
    rest = g_ref[...] - fh_ref[...]
    idx_ref[0] = _vscan(_norm_rows(rest), wn_ref)


def _nearest_codes(si, g, f_hat, Wn, d_mat):
    pl_ = _PATCH[si]
    n = _B * pl_
    if si == len(_PATCH) - 1:
        idx = pl.pallas_call(
            _argmax_last_body,
            grid=(n // _RC,),
            in_specs=[pl.BlockSpec((_RC, _C), lambda c: (c, 0)),
                      pl.BlockSpec((_RC, _C), lambda c: (c, 0)),
                      pl.BlockSpec((_V, _C), lambda c: (0, 0))],
            out_specs=pl.BlockSpec((1, 1, _RC), lambda c: (c, 0, 0)),
            out_shape=jax.ShapeDtypeStruct((n // _RC, 1, _RC), jnp.int32),
        )(g.reshape(n, _C), f_hat.reshape(n, _C), Wn)
    elif pl_ >= _RC:
        idx = pl.pallas_call(
            _argmax_mid_body,
            grid=(_B,),
            in_specs=[pl.BlockSpec((1, _L, _C), lambda b: (b, 0, 0)),
                      pl.BlockSpec((1, _L, _C), lambda b: (b, 0, 0)),
                      pl.BlockSpec((pl_, _L), lambda b: (0, 0)),
                      pl.BlockSpec((_V, _C), lambda b: (0, 0))],
            out_specs=pl.BlockSpec((1, 1, pl_), lambda b: (b, 0, 0)),
            out_shape=jax.ShapeDtypeStruct((_B, 1, pl_), jnp.int32),
        )(g, f_hat, d_mat, Wn)
    else:
        first = si == 0
        in_specs = [pl.BlockSpec((_B, _L, _C), lambda: (0, 0, 0))]
        args = [g]
        if not first:
            in_specs.append(pl.BlockSpec((_B, _L, _C), lambda: (0, 0, 0)))
            args.append(f_hat)
        in_specs += [pl.BlockSpec((pl_, _L), lambda: (0, 0)),
                     pl.BlockSpec((_V, _C), lambda: (0, 0))]
        args += [d_mat, Wn]
        idx = pl.pallas_call(
            functools.partial(_argmax_small_body, pl_, first),
            in_specs=in_specs,
            out_specs=pl.BlockSpec((1, 1, max(256, n)), lambda: (0, 0, 0)),
            out_shape=jax.ShapeDtypeStruct((1, 1, max(256, n)), jnp.int32),
        )(*args)
        return idx.reshape(max(256, n))
    return idx.reshape(n)


def _interp_lohi(l_in, l_out):
    scale = l_in / l_out
    coords = (np.arange(l_out, dtype=np.float32) + np.float32(0.5)) * np.float32(scale) - np.float32(0.5)
    coords = np.clip(coords, 0.0, l_in - 1).astype(np.float32)
    lo = np.floor(coords).astype(np.int32)
    hi = np.minimum(lo + 1, l_in - 1)
    w = (coords - lo.astype(np.float32)).astype(np.float32)
    return lo, hi, w


def _runs(vals):
    out = []
    for v in vals:
        if out and out[-1][0] == v:
            out[-1][1] += 1
        else:
            out.append([int(v), 1])
    return out


def _bcast_rows(hb, runs):
    return jnp.concatenate(
        [jnp.broadcast_to(hb[v:v + 1], (cnt, _C)) for v, cnt in runs], axis=0)


def _onehot_gather(bi, w1_ref, w2_ref, w3_ref):
    n = bi.shape[1]
    io0 = lax.broadcasted_iota(jnp.int32, (_VT, n), 0)
    h = None
    for i in range(_V // _VT):
        oh = jnp.where(io0 == (bi - i * _VT), 1.0, 0.0).astype(jnp.bfloat16)
        sl = slice(i * _VT, (i + 1) * _VT)
        part = lax.dot_general(oh, w1_ref[sl, :], (((0,), (0,)), ((), ())),
                               preferred_element_type=jnp.float32)
        part = part + lax.dot_general(oh, w2_ref[sl, :], (((0,), (0,)), ((), ())),
                                      preferred_element_type=jnp.float32)
        part = part + lax.dot_general(
            oh.astype(jnp.float32), w3_ref[sl, :], (((0,), (0,)), ((), ())),
            preferred_element_type=jnp.float32)
        h = part if h is None else h + part
    return h


def _mono_body(si, *refs):
    pl_ = _PATCH[si]
    first = si == 0
    i = 0
    g_ref = refs[i]; i += 1
    if not first:
        fh_ref = refs[i]; i += 1
        loss_ref = refs[i]; i += 1
    d_ref = refs[i]; i += 1
    wn_ref = refs[i]; i += 1
    w1_ref = refs[i]; i += 1
    w2_ref = refs[i]; i += 1
    w3_ref = refs[i]; i += 1
    if si == 1:
        wa_ref = refs[i]; i += 1
        wb_ref = refs[i]; i += 1
    elif si == 2:
        u_ref = refs[i]; i += 1
    k_ref = refs[i]; i += 1
    b_ref = refs[i]; i += 1
    fh_out, loss_out = refs[i], refs[i + 1]

    d = d_ref[...]
    parts = []
    for b in range(_B):
        rest = g_ref[b] if first else g_ref[b] - fh_ref[b]
        parts.append(_dot3(d, rest))
    rp = jnp.concatenate(parts, axis=0)
    bi = _vscan(_norm_rows(rp), wn_ref)
    h = _onehot_gather(bi, w1_ref, w2_ref, w3_ref)

    if si == 1:
        lo, hi, _ = _interp_lohi(pl_, _L)
        lo_runs, hi_runs = _runs(lo), _runs(hi)
    z = jnp.zeros((1, _C), jnp.float32)
    for b in range(_B):
        hb = h[b * pl_:(b + 1) * pl_]
        if si == 0:
            hu = jnp.broadcast_to(hb[0:1], (_L, _C))
        elif si == 1:
            hu = (wa_ref[...] * _bcast_rows(hb, lo_runs)
                  + wb_ref[...] * _bcast_rows(hb, hi_runs))
        else:
            hu = _dot3(u_ref[...], hb)
        prev = jnp.concatenate([z, hu[:-1]], axis=0)
        nxt = jnp.concatenate([hu[1:], z], axis=0)
        y = (jnp.dot(prev, k_ref[0], preferred_element_type=jnp.float32)
             + jnp.dot(hu, k_ref[1], preferred_element_type=jnp.float32)
             + jnp.dot(nxt, k_ref[2], preferred_element_type=jnp.float32)
             + b_ref[...])
        ho = (1.0 - _RESI) * hu + _RESI * y
        fh = ho if first else fh_ref[b] + ho
        fh_out[b] = fh
        dd = fh - g_ref[b]
        part = jnp.sum(dd * dd).reshape(1, 1)
        loss_out[b] = part if first else loss_ref[b] + part


def _mono_scale(si, g, f_hat, loss, d_mat, Wn, W1, W2, W3, u_mat, k_mats, bias):
    pl_ = _PATCH[si]
    first = si == 0
    full3 = pl.BlockSpec((_B, _L, _C), lambda: (0, 0, 0))
    in_specs = [full3]
    args = [g]
    aliases = {}
    if not first:
        in_specs += [full3, pl.BlockSpec((_B, 1, 1), lambda: (0, 0, 0))]
        args += [f_hat, loss]
    in_specs += [pl.BlockSpec((pl_, _L), lambda: (0, 0)),
                 pl.BlockSpec((_V, _C), lambda: (0, 0)),
                 pl.BlockSpec((_V, _C), lambda: (0, 0)),
                 pl.BlockSpec((_V, _C), lambda: (0, 0)),
                 pl.BlockSpec((_V, _C), lambda: (0, 0))]
    args += [d_mat, Wn, W1, W2, W3]
    if si == 1:
        lo, hi, w = _interp_lohi(pl_, _L)
        wa = jnp.asarray((1.0 - w).reshape(_L, 1))
        wb = jnp.asarray(w.reshape(_L, 1))
        in_specs += [pl.BlockSpec((_L, 1), lambda: (0, 0)),
                     pl.BlockSpec((_L, 1), lambda: (0, 0))]
        args += [wa, wb]
    elif si == 2:
        in_specs.append(pl.BlockSpec((_L, pl_), lambda: (0, 0)))
        args.append(u_mat)
    in_specs += [pl.BlockSpec((3, _C, _C), lambda: (0, 0, 0)),
                 pl.BlockSpec((1, _C), lambda: (0, 0))]
    args += [k_mats, bias]
    if not first:
        aliases = {1: 0, 2: 1}
    return pl.pallas_call(
        functools.partial(_mono_body, si),
        in_specs=in_specs,
        out_specs=[full3, pl.BlockSpec((_B, 1, 1), lambda: (0, 0, 0))],
        out_shape=[jax.ShapeDtypeStruct((_B, _L, _C), jnp.float32),
                   jax.ShapeDtypeStruct((_B, 1, 1), jnp.float32)],
        input_output_aliases=aliases,
    )(*args)


@functools.cache
def _sc_gather_fn(npad):
    info = plsc.get_sparse_core_info()
    nw = info.num_cores * info.num_subcores
    bpw = npad // nw
    mesh = plsc.VectorSubcoreMesh(core_axis_name="c", subcore_axis_name="s")

    @functools.partial(
        pl.kernel,
        mesh=mesh,
        out_type=jax.ShapeDtypeStruct((npad, _CP), jnp.float32),
        scratch_types=[
            pltpu.VMEM((bpw,), jnp.int32),
            pltpu.VMEM((bpw, _CP), jnp.float32),
            pltpu.SemaphoreType.DMA,
        ],
    )
    def gather(table_hbm, idx_hbm, out_hbm, idx_v, rows_v, sem):
        wid = lax.axis_index("s") * info.num_cores + lax.axis_index("c")
        base = wid * bpw
        pltpu.sync_copy(idx_hbm.at[pl.ds(base, bpw)], idx_v)
        pltpu.async_copy(table_hbm.at[idx_v], rows_v, sem).wait()
        pltpu.sync_copy(rows_v, out_hbm.at[pl.ds(base, bpw)])

    return gather


def _gather_rows(Wp, idx):
    n = idx.shape[0]
    npad = max(256, n)
    if npad != n:
        idx = jnp.pad(idx, (0, npad - n))
    rows = _sc_gather_fn(npad)(Wp, idx)
    return rows[:n]


_CB = 4


def _compose_body(si, first, *refs):
    last = si == len(_PATCH) - 1
    i = 0
    h_ref = refs[i]; i += 1
    if not last:
        u_ref = refs[i]; i += 1
    k_ref = refs[i]; i += 1
    b_ref = refs[i]; i += 1
    g_ref = refs[i]; i += 1
    if not first:
        fh_ref = refs[i]; i += 1
        loss_ref = refs[i]; i += 1
    fh_out, loss_out = refs[i], refs[i + 1]

    z = jnp.zeros((1, _C), jnp.float32)
    for bb in range(_CB):
        hp = h_ref[bb][:, :_C]
        if last:
            hu = hp
        elif si == 1:
            hu = jnp.dot(u_ref[...], hp, preferred_element_type=jnp.float32,
                         precision=_HI)
        else:
            hu = _dot3(u_ref[...], hp)
        prev = jnp.concatenate([z, hu[:-1]], axis=0)
        nxt = jnp.concatenate([hu[1:], z], axis=0)
        y = (jnp.dot(prev, k_ref[0], preferred_element_type=jnp.float32)
             + jnp.dot(hu, k_ref[1], preferred_element_type=jnp.float32)
             + jnp.dot(nxt, k_ref[2], preferred_element_type=jnp.float32)
             + b_ref[...])
        ho = (1.0 - _RESI) * hu + _RESI * y
        fh = ho if first else fh_ref[bb] + ho
        if last:
            fh_out[bb] = jnp.transpose(fh, (1, 0))
        else:
            fh_out[bb] = fh
        d = fh - g_ref[bb]
        part = jnp.sum(d * d).reshape(1, 1)
        loss_out[bb] = part if first else loss_ref[bb] + part


def _compose_scale(si, h_rows, u_mat, k_mats, bias, g, f_hat, loss):
    pl_ = _PATCH[si]
    last = si == len(_PATCH) - 1
    first = si == 0
    h = h_rows.reshape(_B, pl_, _CP)
    in_specs = [pl.BlockSpec((_CB, pl_, _CP), lambda b: (b, 0, 0))]
    args = [h]
    if not last:
        in_specs.append(pl.BlockSpec((_L, pl_), lambda b: (0, 0)))
        args.append(u_mat)
    in_specs += [
        pl.BlockSpec((3, _C, _C), lambda b: (0, 0, 0)),
        pl.BlockSpec((1, _C), lambda b: (0, 0)),
        pl.BlockSpec((_CB, _L, _C), lambda b: (b, 0, 0)),
    ]
    args += [k_mats, bias, g]
    aliases = {}
    if not first:
        in_specs += [pl.BlockSpec((_CB, _L, _C), lambda b: (b, 0, 0)),
                     pl.BlockSpec((_CB, 1, 1), lambda b: (b, 0, 0))]
        args += [f_hat, loss]
        if last:
            aliases = {len(args) - 1: 1}
        else:
            aliases = {len(args) - 2: 0, len(args) - 1: 1}
    fh_spec = (pl.BlockSpec((_CB, _C, _L), lambda b: (b, 0, 0)) if last
               else pl.BlockSpec((_CB, _L, _C), lambda b: (b, 0, 0)))
    fh_shape = ((_B, _C, _L) if last else (_B, _L, _C))
    out = pl.pallas_call(
        functools.partial(_compose_body, si, first),
        grid=(_B // _CB,),
        in_specs=in_specs,
        out_specs=[
            fh_spec,
            pl.BlockSpec((_CB, 1, 1), lambda b: (b, 0, 0)),
        ],
        out_shape=[
            jax.ShapeDtypeStruct(fh_shape, jnp.float32),
            jax.ShapeDtypeStruct((_B, 1, 1), jnp.float32),
        ],
        input_output_aliases=aliases,
    )(*args)
    return out


def _loss_total_body(loss_ref, out_ref):
    s = jnp.sum(loss_ref[...])
    scale = (1.0 + _BETA) / (len(_PATCH) * _B * _C * _L)
    out_ref[...] = (s * scale).reshape(1, 1)


def _loss_total(loss):
    out = pl.pallas_call(
        _loss_total_body,
        in_specs=[pl.BlockSpec((_B, 1, 1), lambda: (0, 0, 0))],
        out_specs=pl.BlockSpec((1, 1), lambda: (0, 0)),
        out_shape=jax.ShapeDtypeStruct((1, 1), jnp.float32),
    )(loss)
    return out[0, 0]


def kernel(f_BCL, W, phi_w, phi_b):
    sn = len(_PATCH)
    g = jnp.transpose(f_BCL, (0, 2, 1))
    Wn, Wp, W1, W2, W3 = _prep_codebook(W)

    d_mats = [jnp.asarray(_interp_matrix(_L, p)) for p in _PATCH[:-1]]
    u_mats = [jnp.asarray(_interp_matrix(p, _L)) for p in _PATCH[:-1]]
    k_all = jnp.transpose(phi_w, (0, 3, 2, 1))

    f_hat = None
    loss = None
    for si in range(3):
        k = _PHI_IDX[si]
        f_hat, loss = _mono_scale(
            si, g, f_hat, loss, d_mats[si], Wn, W1, W2, W3,
            u_mats[si], k_all[k], phi_b[k].reshape(1, _C))
    for si in range(3, sn):
        idx = _nearest_codes(si, g, f_hat, Wn,
                             d_mats[si] if si < sn - 1 else None)
        h_rows = _gather_rows(Wp, idx)[:_B * _PATCH[si]]
        k = _PHI_IDX[si]
        f_hat, loss = _compose_scale(
            si, h_rows, u_mats[si] if si < sn - 1 else None,
            k_all[k], phi_b[k].reshape(1, _C), g, f_hat, loss)

    return f_hat, _loss_total(loss)
